# baseline (speedup 1.0000x reference)
import functools

import jax
import jax.numpy as jnp
from jax.experimental import pallas as pl
from jax.experimental.pallas import tpu as pltpu


def _proj_body(x_ref, w_ref, seq_ref):
    seq_ref[...] = jax.lax.dot_general(
        x_ref[...], w_ref[...],
        dimension_numbers=(((1,), (1,)), ((), ())),
        preferred_element_type=jnp.float32,
    )


def _agg_body(seq_ref, b_ref, a_ref, adj_ref, out_ref):
    agg = jnp.dot(adj_ref[...], seq_ref[...], preferred_element_type=jnp.float32)
    agg = agg + b_ref[...]
    out_ref[...] = jnp.where(agg >= 0, agg, a_ref[0, 0] * agg)


def kernel(input, adj, W, bias_1, prelu_a):
    N, IN_F = input.shape
    OUT_F = W.shape[0]
    BM = 200
    assert N % BM == 0

    seq = pl.pallas_call(
        _proj_body,
        out_shape=jax.ShapeDtypeStruct((N, OUT_F), jnp.float32),
    )(input, W)

    bias2d = bias_1.reshape(1, OUT_F)
    a2d = jnp.asarray(prelu_a, jnp.float32).reshape(1, 1)

    return pl.pallas_call(
        _agg_body,
        grid=(N // BM,),
        in_specs=[
            pl.BlockSpec((N, OUT_F), lambda i: (0, 0)),
            pl.BlockSpec((1, OUT_F), lambda i: (0, 0)),
            pl.BlockSpec((1, 1), lambda i: (0, 0)),
            pl.BlockSpec((BM, N), lambda i: (i, 0)),
        ],
        out_specs=pl.BlockSpec((BM, OUT_F), lambda i: (i, 0)),
        out_shape=jax.ShapeDtypeStruct((N, OUT_F), jnp.float32),
        compiler_params=pltpu.CompilerParams(
            dimension_semantics=("parallel",),
        ),
    )(seq, bias2d, a2d, adj)


# speedup vs baseline: 1.0605x; 1.0605x over previous
import jax
import jax.numpy as jnp
from jax.experimental import pallas as pl
from jax.experimental.pallas import tpu as pltpu


def _gconv_body(x_ref, w_ref, b_ref, a_ref, adj_ref, out_ref, seq_ref):
    @pl.when(pl.program_id(0) == 0)
    def _():
        seq_ref[...] = jax.lax.dot_general(
            x_ref[...], w_ref[...],
            dimension_numbers=(((1,), (1,)), ((), ())),
            preferred_element_type=jnp.float32,
        )

    agg = jnp.dot(
        adj_ref[...].astype(jnp.bfloat16),
        seq_ref[...].astype(jnp.bfloat16),
        preferred_element_type=jnp.float32,
    )
    agg = agg + b_ref[...]
    out_ref[...] = jnp.where(agg >= 0, agg, a_ref[0, 0] * agg)


def kernel(input, adj, W, bias_1, prelu_a):
    N, IN_F = input.shape
    OUT_F = W.shape[0]
    BM = 200
    assert N % BM == 0

    bias2d = bias_1.reshape(1, OUT_F)
    a2d = jnp.asarray(prelu_a, jnp.float32).reshape(1, 1)

    return pl.pallas_call(
        _gconv_body,
        grid=(N // BM,),
        in_specs=[
            pl.BlockSpec((N, IN_F), lambda i: (0, 0)),
            pl.BlockSpec((OUT_F, IN_F), lambda i: (0, 0)),
            pl.BlockSpec((1, OUT_F), lambda i: (0, 0)),
            pl.BlockSpec((1, 1), lambda i: (0, 0)),
            pl.BlockSpec((BM, N), lambda i: (i, 0)),
        ],
        out_specs=pl.BlockSpec((BM, OUT_F), lambda i: (i, 0)),
        out_shape=jax.ShapeDtypeStruct((N, OUT_F), jnp.float32),
        scratch_shapes=[pltpu.VMEM((N, OUT_F), jnp.float32)],
    )(input, W, bias2d, a2d, adj)
